# Initial kernel scaffold; baseline (speedup 1.0000x reference)
import jax
import jax.numpy as jnp
from jax.experimental import pallas as pl


def kernel(user_id, artists_id, categories_id, user_factors, entity_factors, relation_k):
    raise NotImplementedError("write your pallas kernel here")


# speedup vs baseline: 1.9669x; 1.9669x over previous
import functools

import jax
import jax.numpy as jnp
from jax import lax
from jax.experimental import pallas as pl
from jax.experimental.pallas import tpu as pltpu
from jax.experimental.pallas import tpu_sc as plsc

B = 16384
L = 50
LP = 56
D = 64
NB = 8
NW = 32
UPW = B // NW
NBLK = UPW // NB


def _body(uid_hbm, ida_hbm, idc_hbm, uf_hbm, ef_hbm, rkt_hbm,
          pred_hbm, sco_hbm, ca_hbm, cd_hbm, na_hbm, nc_hbm,
          uid_v, idxa_v, idxc_v, users_v, rows_a, rows_c, rkt_v,
          niua_v, niuc_v, ca_v, cd_v, sk_v, pred_v, cam_v, cdm_v, sco_v,
          sem):
    wid = lax.axis_index("s") * 2 + lax.axis_index("c")
    base = wid * UPW

    pltpu.sync_copy(rkt_hbm, rkt_v)

    lane = lax.iota(jnp.int32, 16)
    last = lane == 15
    perms = [lane ^ s for s in (8, 4, 2, 1)]

    dnums = lax.GatherDimensionNumbers(
        offset_dims=(), collapsed_slice_dims=(0,), start_index_map=(0,))

    def xlane(v, p):
        return lax.gather(v, p[:, None], dnums, (1,),
                          mode=lax.GatherScatterMode.PROMISE_IN_BOUNDS)

    def vreduce(v):
        for p in perms:
            v = v + xlane(v, p)
        return v

    def scat1(ref, idx, vec):
        plsc.store_scatter(ref, [jnp.full((16,), idx, jnp.int32)], vec,
                           mask=last)

    def block(blk, carry):
        gb = base + blk * NB
        pltpu.sync_copy(uid_hbm.at[pl.ds(gb, NB)], uid_v)
        pltpu.sync_copy(ida_hbm.at[pl.ds(gb, NB), :], idxa_v)
        pltpu.sync_copy(idc_hbm.at[pl.ds(gb, NB), :], idxc_v)
        cps = [pltpu.async_copy(uf_hbm.at[uid_v], users_v, sem)]
        for u in range(NB):
            cps.append(pltpu.async_copy(ef_hbm.at[idxa_v.at[u]], rows_a.at[u], sem))
            cps.append(pltpu.async_copy(ef_hbm.at[idxc_v.at[u]], rows_c.at[u], sem))
        for c in cps:
            c.wait()

        for u in range(NB):
            uv = [users_v[u, pl.ds(16 * k, 16)] for k in range(4)]
            for k3 in range(3):
                acc = rkt_v[k3, pl.ds(0, 16)] * uv[0]
                for k in range(1, 4):
                    acc = acc + rkt_v[k3, pl.ds(16 * k, 16)] * uv[k]
                scat1(sk_v, k3 * UPW + blk * NB + u, vreduce(acc))

            def dots(l, vacc, rows, niu):
                acc = rows[u, l, pl.ds(0, 16)] * uv[0]
                for k in range(1, 4):
                    acc = acc + rows[u, l, pl.ds(16 * k, 16)] * uv[k]
                scat1(niu, u * L + l, vreduce(acc))
                return vacc + acc

            zero = jnp.zeros((16,), jnp.float32)
            va = lax.fori_loop(0, L, lambda l, c: dots(l, c, rows_a, niua_v), zero)
            vc = lax.fori_loop(0, L, lambda l, c: dots(l, c, rows_c, niuc_v), zero)
            scat1(ca_v, blk * NB + u, vreduce(va))
            scat1(cd_v, blk * NB + u, vreduce(vc))

        pltpu.sync_copy(niua_v, na_hbm.at[pl.ds(gb * L, NB * L)])
        pltpu.sync_copy(niuc_v, nc_hbm.at[pl.ds(gb * L, NB * L)])
        return carry

    lax.fori_loop(0, NBLK, block, jnp.int32(0))

    def lrelu(x):
        return jnp.where(x >= 0, x, 0.2 * x)

    for g in range(UPW // 16):
        sl = pl.ds(g * 16, 16)
        s0 = lrelu(sk_v[pl.ds(0 * UPW + g * 16, 16)])
        s1 = lrelu(sk_v[pl.ds(1 * UPW + g * 16, 16)])
        s2 = lrelu(sk_v[pl.ds(2 * UPW + g * 16, 16)])
        m = jnp.maximum(jnp.maximum(s0, s1), s2)
        e0, e1, e2 = jnp.exp(s0 - m), jnp.exp(s1 - m), jnp.exp(s2 - m)
        rden = 1.0 / (e0 + e1 + e2)
        sc0, sc1, sc2 = e0 * rden, e1 * rden, e2 * rden
        ca = ca_v[sl] * jnp.float32(1.0 / L)
        cd = cd_v[sl] * jnp.float32(1.0 / L)
        pred_v[sl] = (ca * sc0 + cd * sc1) / (sc0 + sc1)
        cam_v[sl] = ca
        cdm_v[sl] = cd
        rows3 = (lane + g * 16) * 3
        for k3, sck in enumerate((sc0, sc1, sc2)):
            plsc.store_scatter(sco_v, [rows3 + k3], sck)

    pltpu.sync_copy(pred_v, pred_hbm.at[pl.ds(base, UPW)])
    pltpu.sync_copy(sco_v, sco_hbm.at[pl.ds(base * 3, UPW * 3)])
    pltpu.sync_copy(cam_v, ca_hbm.at[pl.ds(base, UPW)])
    pltpu.sync_copy(cdm_v, cd_hbm.at[pl.ds(base, UPW)])


@jax.jit
def _run(user_id, ida_p, idc_p, user_factors, entity_factors, rkt):
    f32 = jnp.float32
    out_type = [
        jax.ShapeDtypeStruct((B,), f32),
        jax.ShapeDtypeStruct((B * 3,), f32),
        jax.ShapeDtypeStruct((B,), f32),
        jax.ShapeDtypeStruct((B,), f32),
        jax.ShapeDtypeStruct((B * L,), f32),
        jax.ShapeDtypeStruct((B * L,), f32),
    ]
    scratch_types = [
        pltpu.VMEM((NB,), jnp.int32),
        pltpu.VMEM((NB, LP), jnp.int32),
        pltpu.VMEM((NB, LP), jnp.int32),
        pltpu.VMEM((NB, D), f32),
        pltpu.VMEM((NB, LP, D), f32),
        pltpu.VMEM((NB, LP, D), f32),
        pltpu.VMEM((3, D), f32),
        pltpu.VMEM((NB * L,), f32),
        pltpu.VMEM((NB * L,), f32),
        pltpu.VMEM((UPW,), f32),
        pltpu.VMEM((UPW,), f32),
        pltpu.VMEM((3 * UPW,), f32),
        pltpu.VMEM((UPW,), f32),
        pltpu.VMEM((UPW,), f32),
        pltpu.VMEM((UPW,), f32),
        pltpu.VMEM((UPW * 3,), f32),
        pltpu.SemaphoreType.DMA,
    ]
    mesh = plsc.VectorSubcoreMesh(core_axis_name="c", subcore_axis_name="s")
    fn = pl.kernel(_body, mesh=mesh, out_type=out_type,
                   scratch_types=scratch_types,
                   compiler_params=pltpu.CompilerParams(
                       needs_layout_passes=False,
                       use_tc_tiling_on_sc=False))
    return fn(user_id, ida_p, idc_p, user_factors, entity_factors, rkt)


def kernel(user_id, artists_id, categories_id, user_factors, entity_factors,
           relation_k):
    uid = user_id.astype(jnp.int32)
    ida = jnp.pad(artists_id.astype(jnp.int32), ((0, 0), (0, LP - L)))
    idc = jnp.pad(categories_id.astype(jnp.int32), ((0, 0), (0, LP - L)))
    rkt = relation_k.T
    pred, sco, ca, cd, na, nc = _run(uid, ida, idc, user_factors,
                                     entity_factors, rkt)
    return (pred, sco.reshape(B, 3), ca, cd,
            (na.reshape(B, L, 1), nc.reshape(B, L, 1)))


# speedup vs baseline: 6.5821x; 3.3465x over previous
import functools

import jax
import jax.numpy as jnp
from jax import lax
from jax.experimental import pallas as pl
from jax.experimental.pallas import tpu as pltpu
from jax.experimental.pallas import tpu_sc as plsc

B = 16384
L = 50
D = 64
NB = 8
RPB = NB * L
NW = 32
UPW = B // NW
NBLK = UPW // NB
CHUNKS = ((0, 104), (104, 104), (208, 104), (312, 88))


def _body(uid_hbm, ida_hbm, idc_hbm, uf_hbm, ef_hbm, rkt_hbm,
          pred_hbm, sco_hbm, ca_hbm, cd_hbm, na_hbm, nc_hbm,
          uid_v0, uid_v1, idxa_v0, idxa_v1, idxc_v0, idxc_v1,
          users_v0, users_v1, rows_a0, rows_a1, rows_c0, rows_c1,
          niua_v0, niua_v1, niuc_v0, niuc_v1,
          rkt_v, ca_v, cd_v, sk_v, pred_v, cam_v, cdm_v, sco_v,
          sem_g0, sem_g1, sem_o0, sem_o1):
    wid = lax.axis_index("s") * 2 + lax.axis_index("c")
    base = wid * UPW

    slots = (
        (uid_v0, idxa_v0, idxc_v0, users_v0, rows_a0, rows_c0,
         niua_v0, niuc_v0, sem_g0, sem_o0),
        (uid_v1, idxa_v1, idxc_v1, users_v1, rows_a1, rows_c1,
         niua_v1, niuc_v1, sem_g1, sem_o1),
    )

    pltpu.sync_copy(rkt_hbm, rkt_v)

    lane = lax.iota(jnp.int32, 16)
    last = lane == 15
    perms = [lane ^ s for s in (8, 4, 2, 1)]

    dnums = lax.GatherDimensionNumbers(
        offset_dims=(), collapsed_slice_dims=(0,), start_index_map=(0,))

    def xlane(v, p):
        return lax.gather(v, p[:, None], dnums, (1,),
                          mode=lax.GatherScatterMode.PROMISE_IN_BOUNDS)

    def vreduce(v):
        for p in perms:
            v = v + xlane(v, p)
        return v

    def scat1(ref, idx, vec):
        plsc.store_scatter(ref, [jnp.full((16,), idx, jnp.int32)], vec,
                           mask=last)

    def stage(slot, blk):
        uid_v, idxa_v, idxc_v, users_v, rows_a, rows_c, _, _, sem_g, _ = slot
        gb = base + blk * NB
        pltpu.sync_copy(uid_hbm.at[pl.ds(gb, NB)], uid_v)
        pltpu.sync_copy(ida_hbm.at[pl.ds(gb * L, RPB)], idxa_v)
        pltpu.sync_copy(idc_hbm.at[pl.ds(gb * L, RPB)], idxc_v)
        pltpu.async_copy(uf_hbm.at[uid_v], users_v, sem_g)
        for off, n in CHUNKS:
            pltpu.async_copy(ef_hbm.at[idxa_v.at[pl.ds(off, n)]],
                             rows_a.at[pl.ds(off, n)], sem_g)
            pltpu.async_copy(ef_hbm.at[idxc_v.at[pl.ds(off, n)]],
                             rows_c.at[pl.ds(off, n)], sem_g)

    def wait_gathers(slot):
        _, _, _, users_v, rows_a, rows_c, _, _, sem_g, _ = slot
        pltpu.make_async_copy(uf_hbm.at[pl.ds(0, NB)], users_v, sem_g).wait()
        pltpu.make_async_copy(ef_hbm.at[pl.ds(0, RPB)], rows_a, sem_g).wait()
        pltpu.make_async_copy(ef_hbm.at[pl.ds(0, RPB)], rows_c, sem_g).wait()

    def compute(slot, blk):
        _, _, _, users_v, rows_a, rows_c, niua_v, niuc_v, _, sem_o = slot
        gb = base + blk * NB
        @pl.when(blk >= 2)
        def _():
            pltpu.make_async_copy(na_hbm.at[pl.ds(0, RPB)], niua_v,
                                  sem_o).wait()
            pltpu.make_async_copy(nc_hbm.at[pl.ds(0, RPB)], niuc_v,
                                  sem_o).wait()

        def user_body(u, carry):
            uv = [users_v[u, pl.ds(16 * k, 16)] for k in range(4)]
            for k3 in range(3):
                acc = rkt_v[k3, pl.ds(0, 16)] * uv[0]
                for k in range(1, 4):
                    acc = acc + rkt_v[k3, pl.ds(16 * k, 16)] * uv[k]
                scat1(sk_v, k3 * UPW + blk * NB + u, vreduce(acc))

            def dots(l, vacc, rows, niu):
                acc = rows[u * L + l, pl.ds(0, 16)] * uv[0]
                for k in range(1, 4):
                    acc = acc + rows[u * L + l, pl.ds(16 * k, 16)] * uv[k]
                scat1(niu, u * L + l, vreduce(acc))
                return vacc + acc

            zero = jnp.zeros((16,), jnp.float32)
            va = lax.fori_loop(0, L, lambda l, c: dots(l, c, rows_a, niua_v),
                               zero, unroll=10)
            vc = lax.fori_loop(0, L, lambda l, c: dots(l, c, rows_c, niuc_v),
                               zero, unroll=10)
            scat1(ca_v, blk * NB + u, vreduce(va))
            scat1(cd_v, blk * NB + u, vreduce(vc))
            return carry

        lax.fori_loop(0, NB, user_body, jnp.int32(0))

        pltpu.async_copy(niua_v, na_hbm.at[pl.ds(gb * L, RPB)], sem_o)
        pltpu.async_copy(niuc_v, nc_hbm.at[pl.ds(gb * L, RPB)], sem_o)

    stage(slots[0], 0)

    def pair(gi, carry):
        for p in range(2):
            blk = gi * 2 + p

            @pl.when(blk + 1 < NBLK)
            def _():
                stage(slots[1 - p], blk + 1)

            wait_gathers(slots[p])
            compute(slots[p], blk)
        return carry

    lax.fori_loop(0, NBLK // 2, pair, jnp.int32(0))

    pltpu.make_async_copy(na_hbm.at[pl.ds(0, RPB)], niua_v0, sem_o0).wait()
    pltpu.make_async_copy(nc_hbm.at[pl.ds(0, RPB)], niuc_v0, sem_o0).wait()
    pltpu.make_async_copy(na_hbm.at[pl.ds(0, RPB)], niua_v1, sem_o1).wait()
    pltpu.make_async_copy(nc_hbm.at[pl.ds(0, RPB)], niuc_v1, sem_o1).wait()

    def lrelu(x):
        return jnp.where(x >= 0, x, 0.2 * x)

    def head(g, carry):
        sl = pl.ds(g * 16, 16)
        s0 = lrelu(sk_v[pl.ds(0 * UPW + g * 16, 16)])
        s1 = lrelu(sk_v[pl.ds(1 * UPW + g * 16, 16)])
        s2 = lrelu(sk_v[pl.ds(2 * UPW + g * 16, 16)])
        m = jnp.maximum(jnp.maximum(s0, s1), s2)
        e0, e1, e2 = jnp.exp(s0 - m), jnp.exp(s1 - m), jnp.exp(s2 - m)
        rden = 1.0 / (e0 + e1 + e2)
        sc0, sc1, sc2 = e0 * rden, e1 * rden, e2 * rden
        ca = ca_v[sl] * jnp.float32(1.0 / L)
        cd = cd_v[sl] * jnp.float32(1.0 / L)
        pred_v[sl] = (ca * sc0 + cd * sc1) / (sc0 + sc1)
        cam_v[sl] = ca
        cdm_v[sl] = cd
        rows3 = (lane + g * 16) * 3
        for k3, sck in enumerate((sc0, sc1, sc2)):
            plsc.store_scatter(sco_v, [rows3 + k3], sck)
        return carry

    lax.fori_loop(0, UPW // 16, head, jnp.int32(0), unroll=4)

    pltpu.sync_copy(pred_v, pred_hbm.at[pl.ds(base, UPW)])
    pltpu.sync_copy(sco_v, sco_hbm.at[pl.ds(base * 3, UPW * 3)])
    pltpu.sync_copy(cam_v, ca_hbm.at[pl.ds(base, UPW)])
    pltpu.sync_copy(cdm_v, cd_hbm.at[pl.ds(base, UPW)])


@jax.jit
def _run(user_id, ida_f, idc_f, user_factors, entity_factors, rkt):
    f32 = jnp.float32
    out_type = [
        jax.ShapeDtypeStruct((B,), f32),
        jax.ShapeDtypeStruct((B * 3,), f32),
        jax.ShapeDtypeStruct((B,), f32),
        jax.ShapeDtypeStruct((B,), f32),
        jax.ShapeDtypeStruct((B * L,), f32),
        jax.ShapeDtypeStruct((B * L,), f32),
    ]
    i32 = jnp.int32
    scratch_types = [
        pltpu.VMEM((NB,), i32), pltpu.VMEM((NB,), i32),
        pltpu.VMEM((RPB,), i32), pltpu.VMEM((RPB,), i32),
        pltpu.VMEM((RPB,), i32), pltpu.VMEM((RPB,), i32),
        pltpu.VMEM((NB, D), f32), pltpu.VMEM((NB, D), f32),
        pltpu.VMEM((RPB, D), f32), pltpu.VMEM((RPB, D), f32),
        pltpu.VMEM((RPB, D), f32), pltpu.VMEM((RPB, D), f32),
        pltpu.VMEM((RPB,), f32), pltpu.VMEM((RPB,), f32),
        pltpu.VMEM((RPB,), f32), pltpu.VMEM((RPB,), f32),
        pltpu.VMEM((3, D), f32),
        pltpu.VMEM((UPW,), f32),
        pltpu.VMEM((UPW,), f32),
        pltpu.VMEM((3 * UPW,), f32),
        pltpu.VMEM((UPW,), f32),
        pltpu.VMEM((UPW,), f32),
        pltpu.VMEM((UPW,), f32),
        pltpu.VMEM((UPW * 3,), f32),
        pltpu.SemaphoreType.DMA, pltpu.SemaphoreType.DMA,
        pltpu.SemaphoreType.DMA, pltpu.SemaphoreType.DMA,
    ]
    mesh = plsc.VectorSubcoreMesh(core_axis_name="c", subcore_axis_name="s")
    fn = pl.kernel(_body, mesh=mesh, out_type=out_type,
                   scratch_types=scratch_types,
                   compiler_params=pltpu.CompilerParams(
                       needs_layout_passes=False,
                       use_tc_tiling_on_sc=False))
    return fn(user_id, ida_f, idc_f, user_factors, entity_factors, rkt)


def kernel(user_id, artists_id, categories_id, user_factors, entity_factors,
           relation_k):
    uid = user_id.astype(jnp.int32)
    ida = artists_id.astype(jnp.int32).reshape(-1)
    idc = categories_id.astype(jnp.int32).reshape(-1)
    rkt = relation_k.T
    pred, sco, ca, cd, na, nc = _run(uid, ida, idc, user_factors,
                                     entity_factors, rkt)
    return (pred, sco.reshape(B, 3), ca, cd,
            (na.reshape(B, L, 1), nc.reshape(B, L, 1)))


# speedup vs baseline: 8.7134x; 1.3238x over previous
import functools

import jax
import jax.numpy as jnp
from jax import lax
from jax.experimental import pallas as pl
from jax.experimental.pallas import tpu as pltpu
from jax.experimental.pallas import tpu_sc as plsc

B = 16384
L = 50
D = 64
NB = 8
RPB = NB * L
NW = 32
UPW = B // NW
NBLK = UPW // NB
CHUNKS = ((0, 104), (104, 104), (208, 104), (312, 88))


def _body(uid_hbm, ida_hbm, idc_hbm, uf_hbm, ef_hbm, rkt_hbm,
          pred_hbm, sco_hbm, ca_hbm, cd_hbm, na_hbm, nc_hbm,
          uid_v0, uid_v1, idxa_v0, idxa_v1, idxc_v0, idxc_v1,
          users_v0, users_v1, rows_a0, rows_a1, rows_c0, rows_c1,
          niua_v0, niua_v1, niuc_v0, niuc_v1,
          part_a, part_c,
          rkt_v, ca_v, cd_v, sk_v, pred_v, cam_v, cdm_v, sco_v,
          sem_g0, sem_g1, sem_o0, sem_o1):
    wid = lax.axis_index("s") * 2 + lax.axis_index("c")
    base = wid * UPW

    slots = (
        (uid_v0, idxa_v0, idxc_v0, users_v0, rows_a0, rows_c0,
         niua_v0, niuc_v0, sem_g0, sem_o0),
        (uid_v1, idxa_v1, idxc_v1, users_v1, rows_a1, rows_c1,
         niua_v1, niuc_v1, sem_g1, sem_o1),
    )

    pltpu.sync_copy(rkt_hbm, rkt_v)

    lane = lax.iota(jnp.int32, 16)
    lane16 = lane * 16
    last = lane == 15
    perms = [lane ^ s for s in (8, 4, 2, 1)]

    dnums = lax.GatherDimensionNumbers(
        offset_dims=(), collapsed_slice_dims=(0,), start_index_map=(0,))

    def xlane(v, p):
        return lax.gather(v, p[:, None], dnums, (1,),
                          mode=lax.GatherScatterMode.PROMISE_IN_BOUNDS)

    def vreduce(v):
        for p in perms:
            v = v + xlane(v, p)
        return v

    def scat1(ref, idx, vec):
        plsc.store_scatter(ref, [jnp.full((16,), idx, jnp.int32)], vec,
                           mask=last)

    def stage(slot, blk):
        uid_v, idxa_v, idxc_v, users_v, rows_a, rows_c, _, _, sem_g, _ = slot
        gb = base + blk * NB
        pltpu.sync_copy(uid_hbm.at[pl.ds(gb, NB)], uid_v)
        pltpu.sync_copy(ida_hbm.at[pl.ds(gb * L, RPB)], idxa_v)
        pltpu.sync_copy(idc_hbm.at[pl.ds(gb * L, RPB)], idxc_v)
        pltpu.async_copy(uf_hbm.at[uid_v], users_v, sem_g)
        for off, n in CHUNKS:
            pltpu.async_copy(ef_hbm.at[idxa_v.at[pl.ds(off, n)]],
                             rows_a.at[pl.ds(off, n)], sem_g)
            pltpu.async_copy(ef_hbm.at[idxc_v.at[pl.ds(off, n)]],
                             rows_c.at[pl.ds(off, n)], sem_g)

    def wait_gathers(slot):
        _, _, _, users_v, rows_a, rows_c, _, _, sem_g, _ = slot
        pltpu.make_async_copy(uf_hbm.at[pl.ds(0, NB)], users_v, sem_g).wait()
        pltpu.make_async_copy(ef_hbm.at[pl.ds(0, RPB)], rows_a, sem_g).wait()
        pltpu.make_async_copy(ef_hbm.at[pl.ds(0, RPB)], rows_c, sem_g).wait()

    def compute(slot, blk):
        _, _, _, users_v, rows_a, rows_c, niua_v, niuc_v, _, sem_o = slot
        gb = base + blk * NB
        @pl.when(blk >= 2)
        def _():
            pltpu.make_async_copy(na_hbm.at[pl.ds(0, RPB)], niua_v,
                                  sem_o).wait()
            pltpu.make_async_copy(nc_hbm.at[pl.ds(0, RPB)], niuc_v,
                                  sem_o).wait()

        def table(u, uv, rows, niu, part_v):
            def dots(l, vacc):
                acc01 = (rows[u * L + l, pl.ds(0, 16)] * uv[0]
                         + rows[u * L + l, pl.ds(16, 16)] * uv[1])
                acc23 = (rows[u * L + l, pl.ds(32, 16)] * uv[2]
                         + rows[u * L + l, pl.ds(48, 16)] * uv[3])
                acc = acc01 + acc23
                part_v[pl.ds(l * 16, 16)] = acc
                return vacc + acc

            zero = jnp.zeros((16,), jnp.float32)
            vacc = lax.fori_loop(0, L, dots, zero, unroll=10)

            for g in range(4):
                terms = [plsc.load_gather(part_v, [lane16 + (256 * g + j)])
                         for j in range(16)]
                while len(terms) > 1:
                    terms = [terms[i] + terms[i + 1]
                             for i in range(0, len(terms) - 1, 2)] + (
                                 [terms[-1]] if len(terms) % 2 else [])
                dots16 = terms[0]
                idx = jnp.full((16,), u * L + 16 * g, jnp.int32) + lane
                if g < 3:
                    plsc.store_scatter(niu, [idx], dots16)
                else:
                    plsc.store_scatter(niu, [idx], dots16, mask=lane < 2)
            return vacc

        def user_body(u, carry):
            uv = [users_v[u, pl.ds(16 * k, 16)] for k in range(4)]
            for k3 in range(3):
                acc = rkt_v[k3, pl.ds(0, 16)] * uv[0]
                for k in range(1, 4):
                    acc = acc + rkt_v[k3, pl.ds(16 * k, 16)] * uv[k]
                scat1(sk_v, k3 * UPW + blk * NB + u, vreduce(acc))

            va = table(u, uv, rows_a, niua_v, part_a)
            vc = table(u, uv, rows_c, niuc_v, part_c)
            scat1(ca_v, blk * NB + u, vreduce(va))
            scat1(cd_v, blk * NB + u, vreduce(vc))
            return carry

        lax.fori_loop(0, NB, user_body, jnp.int32(0))

        pltpu.async_copy(niua_v, na_hbm.at[pl.ds(gb * L, RPB)], sem_o)
        pltpu.async_copy(niuc_v, nc_hbm.at[pl.ds(gb * L, RPB)], sem_o)

    stage(slots[0], 0)

    def pair(gi, carry):
        for p in range(2):
            blk = gi * 2 + p

            @pl.when(blk + 1 < NBLK)
            def _():
                stage(slots[1 - p], blk + 1)

            wait_gathers(slots[p])
            compute(slots[p], blk)
        return carry

    lax.fori_loop(0, NBLK // 2, pair, jnp.int32(0))

    pltpu.make_async_copy(na_hbm.at[pl.ds(0, RPB)], niua_v0, sem_o0).wait()
    pltpu.make_async_copy(nc_hbm.at[pl.ds(0, RPB)], niuc_v0, sem_o0).wait()
    pltpu.make_async_copy(na_hbm.at[pl.ds(0, RPB)], niua_v1, sem_o1).wait()
    pltpu.make_async_copy(nc_hbm.at[pl.ds(0, RPB)], niuc_v1, sem_o1).wait()

    def lrelu(x):
        return jnp.where(x >= 0, x, 0.2 * x)

    def head(g, carry):
        sl = pl.ds(g * 16, 16)
        s0 = lrelu(sk_v[pl.ds(0 * UPW + g * 16, 16)])
        s1 = lrelu(sk_v[pl.ds(1 * UPW + g * 16, 16)])
        s2 = lrelu(sk_v[pl.ds(2 * UPW + g * 16, 16)])
        m = jnp.maximum(jnp.maximum(s0, s1), s2)
        e0, e1, e2 = jnp.exp(s0 - m), jnp.exp(s1 - m), jnp.exp(s2 - m)
        rden = 1.0 / (e0 + e1 + e2)
        sc0, sc1, sc2 = e0 * rden, e1 * rden, e2 * rden
        ca = ca_v[sl] * jnp.float32(1.0 / L)
        cd = cd_v[sl] * jnp.float32(1.0 / L)
        pred_v[sl] = (ca * sc0 + cd * sc1) / (sc0 + sc1)
        cam_v[sl] = ca
        cdm_v[sl] = cd
        rows3 = (lane + g * 16) * 3
        for k3, sck in enumerate((sc0, sc1, sc2)):
            plsc.store_scatter(sco_v, [rows3 + k3], sck)
        return carry

    lax.fori_loop(0, UPW // 16, head, jnp.int32(0), unroll=4)

    pltpu.sync_copy(pred_v, pred_hbm.at[pl.ds(base, UPW)])
    pltpu.sync_copy(sco_v, sco_hbm.at[pl.ds(base * 3, UPW * 3)])
    pltpu.sync_copy(cam_v, ca_hbm.at[pl.ds(base, UPW)])
    pltpu.sync_copy(cdm_v, cd_hbm.at[pl.ds(base, UPW)])


@jax.jit
def _run(user_id, ida_f, idc_f, user_factors, entity_factors, rkt):
    f32 = jnp.float32
    out_type = [
        jax.ShapeDtypeStruct((B,), f32),
        jax.ShapeDtypeStruct((B * 3,), f32),
        jax.ShapeDtypeStruct((B,), f32),
        jax.ShapeDtypeStruct((B,), f32),
        jax.ShapeDtypeStruct((B * L,), f32),
        jax.ShapeDtypeStruct((B * L,), f32),
    ]
    i32 = jnp.int32
    scratch_types = [
        pltpu.VMEM((NB,), i32), pltpu.VMEM((NB,), i32),
        pltpu.VMEM((RPB,), i32), pltpu.VMEM((RPB,), i32),
        pltpu.VMEM((RPB,), i32), pltpu.VMEM((RPB,), i32),
        pltpu.VMEM((NB, D), f32), pltpu.VMEM((NB, D), f32),
        pltpu.VMEM((RPB, D), f32), pltpu.VMEM((RPB, D), f32),
        pltpu.VMEM((RPB, D), f32), pltpu.VMEM((RPB, D), f32),
        pltpu.VMEM((RPB,), f32), pltpu.VMEM((RPB,), f32),
        pltpu.VMEM((RPB,), f32), pltpu.VMEM((RPB,), f32),
        pltpu.VMEM((16 * L,), f32),
        pltpu.VMEM((16 * L,), f32),
        pltpu.VMEM((3, D), f32),
        pltpu.VMEM((UPW,), f32),
        pltpu.VMEM((UPW,), f32),
        pltpu.VMEM((3 * UPW,), f32),
        pltpu.VMEM((UPW,), f32),
        pltpu.VMEM((UPW,), f32),
        pltpu.VMEM((UPW,), f32),
        pltpu.VMEM((UPW * 3,), f32),
        pltpu.SemaphoreType.DMA, pltpu.SemaphoreType.DMA,
        pltpu.SemaphoreType.DMA, pltpu.SemaphoreType.DMA,
    ]
    mesh = plsc.VectorSubcoreMesh(core_axis_name="c", subcore_axis_name="s")
    fn = pl.kernel(_body, mesh=mesh, out_type=out_type,
                   scratch_types=scratch_types,
                   compiler_params=pltpu.CompilerParams(
                       needs_layout_passes=False,
                       use_tc_tiling_on_sc=False))
    return fn(user_id, ida_f, idc_f, user_factors, entity_factors, rkt)


def kernel(user_id, artists_id, categories_id, user_factors, entity_factors,
           relation_k):
    uid = user_id.astype(jnp.int32)
    ida = artists_id.astype(jnp.int32).reshape(-1)
    idc = categories_id.astype(jnp.int32).reshape(-1)
    rkt = relation_k.T
    pred, sco, ca, cd, na, nc = _run(uid, ida, idc, user_factors,
                                     entity_factors, rkt)
    return (pred, sco.reshape(B, 3), ca, cd,
            (na.reshape(B, L, 1), nc.reshape(B, L, 1)))
